# baseline (speedup 1.0000x reference)
import jax
import jax.numpy as jnp
from jax.experimental import pallas as pl
from jax.experimental.pallas import tpu as pltpu

_bf16 = jnp.bfloat16
_f32 = jnp.float32


def _bfp(a, b):
    return a.astype(_bf16).astype(_f32) * b.astype(_bf16).astype(_f32)


def _decode_body(rawx_ref, rawy_ref, rawxf_ref, rawyf_ref, fi_ref,
                 ws0_ref, ws1_ref, bs_ref,
                 wd0_ref, wd1_ref, bd_ref, wiT_ref, whT_ref, bi_ref, bh_ref,
                 wrefT_ref, wqT_ref, v_ref,
                 idx_ref, logp_ref,
                 rp_scr, h_scr, mask_scr):
    N, B = rawx_ref.shape
    H = h_scr.shape[0]
    T = N - 1

    wrefT_bf = wrefT_ref[...].astype(_bf16)
    ws0 = ws0_ref[...]
    ws1 = ws1_ref[...]
    bs = bs_ref[...]

    rxf = rawxf_ref[...]
    ryf = rawyf_ref[...]
    feat = _bfp(ws0, rxf) + _bfp(ws1, ryf) + bs
    rp_scr[...] = jnp.dot(wrefT_bf, feat.astype(_bf16),
                          preferred_element_type=_f32)

    h_scr[...] = jnp.zeros((H, B), dtype=_f32)
    iota_i = jax.lax.broadcasted_iota(jnp.int32, (N, B), 0)
    iota_n = iota_i.astype(_f32)
    mask_scr[...] = jnp.where(iota_i == 0, 0.0, 1.0)

    wd0 = wd0_ref[...]
    wd1 = wd1_ref[...]
    bd = bd_ref[...]
    wiT_bf = wiT_ref[...].astype(_bf16)
    whT_bf = whT_ref[...].astype(_bf16)
    bi2 = bi_ref[...]
    bh2 = bh_ref[...]
    wqT_bf = wqT_ref[...].astype(_bf16)
    v_bf = v_ref[...].astype(_bf16)

    def step(t, dec):
        dx, dy = dec
        emb = _bfp(wd0, dx) + _bfp(wd1, dy) + bd
        hT = h_scr[...]
        gi = jnp.dot(wiT_bf, emb.astype(_bf16), preferred_element_type=_f32) + bi2
        gh = jnp.dot(whT_bf, hT.astype(_bf16), preferred_element_type=_f32) + bh2
        r = jax.nn.sigmoid(gi[0:H] + gh[0:H])
        z = jax.nn.sigmoid(gi[H:2 * H] + gh[H:2 * H])
        n_ = jnp.tanh(gi[2 * H:3 * H] + r * gh[2 * H:3 * H])
        hT = (1.0 - z) * n_ + z * hT
        h_scr[...] = hT

        qT = jnp.dot(wqT_bf, hT.astype(_bf16), preferred_element_type=_f32)

        q_rep = jnp.concatenate([qT] * 8, axis=1)
        rows = []
        for g in range(0, N, 8):
            w = min(8, N - g)
            tt = jnp.tanh(rp_scr[:, g * B:(g + w) * B] + q_rep[:, :w * B])
            lg = jnp.dot(v_bf, tt.astype(_bf16), preferred_element_type=_f32)
            rows.extend(lg[:, j * B:(j + 1) * B] for j in range(w))
        logits = jnp.concatenate(rows, axis=0)

        maskv = mask_scr[...]
        masked = jnp.where(maskv > 0.0, logits, -jnp.inf)
        m = jnp.max(masked, axis=0, keepdims=True)
        unnorm = jnp.exp(masked - m)
        s = jnp.sum(unnorm, axis=0, keepdims=True)
        probs = unnorm / s
        prob = jnp.max(probs, axis=0, keepdims=True)
        logp = jnp.log(prob)

        eq = probs == prob
        chosen = jnp.min(jnp.where(eq, iota_n, 1e9), axis=0, keepdims=True)
        onehot = (iota_n == chosen).astype(_f32)
        mask_scr[...] = maskv * (1.0 - onehot)

        idx_ref[pl.ds(t, 1), :] = chosen
        logp_ref[pl.ds(t, 1), :] = logp

        ndx = jnp.sum(onehot * rawx_ref[...], axis=0, keepdims=True)
        ndy = jnp.sum(onehot * rawy_ref[...], axis=0, keepdims=True)
        return (ndx, ndy)

    jax.lax.fori_loop(0, T, step, (fi_ref[0:1, :], fi_ref[1:2, :]))


def kernel(raw_features, first_input, W_s, b_s, W_d, b_d, Wi, Wh, bi, bh, W_ref, W_q, v):
    B, N, F = raw_features.shape
    H = W_s.shape[1]
    T = N - 1

    rawxT = raw_features[:, :, 0].T
    rawyT = raw_features[:, :, 1].T
    rawxF = rawxT.reshape(1, N * B)
    rawyF = rawyT.reshape(1, N * B)
    fi = jnp.broadcast_to(first_input[0, 0, :][:, None], (F, B)).astype(_f32)

    out_idx, out_logp = pl.pallas_call(
        _decode_body,
        out_shape=[
            jax.ShapeDtypeStruct((T, B), _f32),
            jax.ShapeDtypeStruct((T, B), _f32),
        ],
        scratch_shapes=[
            pltpu.VMEM((H, N * B), _f32),
            pltpu.VMEM((H, B), _f32),
            pltpu.VMEM((N, B), _f32),
        ],
    )(rawxT, rawyT, rawxF, rawyF, fi, W_s[0][:, None], W_s[1][:, None], b_s[:, None],
      W_d[0][:, None], W_d[1][:, None], b_d[:, None], Wi.T, Wh.T,
      bi[:, None], bh[:, None], W_ref.T, W_q.T, v[None, :])

    return (out_idx.T, out_logp.T)


# speedup vs baseline: 1.0061x; 1.0061x over previous
import jax
import jax.numpy as jnp
from jax.experimental import pallas as pl
from jax.experimental.pallas import tpu as pltpu

_bf16 = jnp.bfloat16
_f32 = jnp.float32


def _bfp(a, b):
    return a.astype(_bf16).astype(_f32) * b.astype(_bf16).astype(_f32)


def _decode_body(rawx_ref, rawy_ref, rawxf_ref, rawyf_ref, fi_ref,
                 ws0_ref, ws1_ref, bs_ref,
                 wd0_ref, wd1_ref, bd_ref, wiT_ref, whT_ref, bi_ref, bh_ref,
                 wrefT_ref, wqT_ref, v_ref,
                 idx_ref, logp_ref,
                 rp_scr, h_scr, mask_scr):
    N, B = rawx_ref.shape
    H = h_scr.shape[0]
    T = N - 1

    wrefT_bf = wrefT_ref[...]
    ws0 = ws0_ref[...]
    ws1 = ws1_ref[...]
    bs = bs_ref[...]

    rxf = rawxf_ref[...]
    ryf = rawyf_ref[...]
    feat = _bfp(ws0, rxf) + _bfp(ws1, ryf) + bs
    rp_scr[...] = jnp.dot(wrefT_bf, feat, preferred_element_type=_f32)

    h_scr[...] = jnp.zeros((H, B), dtype=_f32)
    iota_i = jax.lax.broadcasted_iota(jnp.int32, (N, B), 0)
    iota_n = iota_i.astype(_f32)
    mask_scr[...] = jnp.where(iota_i == 0, 0.0, 1.0)

    wd0 = wd0_ref[...]
    wd1 = wd1_ref[...]
    bd = bd_ref[...]
    wiT_bf = wiT_ref[...]
    whT_bf = whT_ref[...]
    bi2 = bi_ref[...]
    bh2 = bh_ref[...]
    wqT_bf = wqT_ref[...]
    v_bf = v_ref[...]

    def step(t, dec):
        dx, dy = dec
        emb = _bfp(wd0, dx) + _bfp(wd1, dy) + bd
        hT = h_scr[...]
        gi = jnp.dot(wiT_bf, emb, preferred_element_type=_f32) + bi2
        gh = jnp.dot(whT_bf, hT, preferred_element_type=_f32) + bh2
        r = jax.nn.sigmoid(gi[0:H] + gh[0:H])
        z = jax.nn.sigmoid(gi[H:2 * H] + gh[H:2 * H])
        n_ = jnp.tanh(gi[2 * H:3 * H] + r * gh[2 * H:3 * H])
        hT = (1.0 - z) * n_ + z * hT
        h_scr[...] = hT

        qT = jnp.dot(wqT_bf, hT, preferred_element_type=_f32)

        q_rep = jnp.concatenate([qT] * 8, axis=1)
        rows = []
        for g in range(0, N, 8):
            w = min(8, N - g)
            tt = jnp.tanh(rp_scr[:, g * B:(g + w) * B] + q_rep[:, :w * B])
            lg = jnp.dot(v_bf, tt, preferred_element_type=_f32)
            rows.extend(lg[:, j * B:(j + 1) * B] for j in range(w))
        logits = jnp.concatenate(rows, axis=0)

        maskv = mask_scr[...]
        masked = jnp.where(maskv > 0.0, logits, -jnp.inf)
        m = jnp.max(masked, axis=0, keepdims=True)
        unnorm = jnp.exp(masked - m)
        s = jnp.sum(unnorm, axis=0, keepdims=True)
        probs = unnorm / s
        prob = jnp.max(probs, axis=0, keepdims=True)
        logp = jnp.log(prob)

        eq = probs == prob
        chosen = jnp.min(jnp.where(eq, iota_n, 1e9), axis=0, keepdims=True)
        onehot = (iota_n == chosen).astype(_f32)
        mask_scr[...] = maskv * (1.0 - onehot)

        idx_ref[pl.ds(t, 1), :] = chosen
        logp_ref[pl.ds(t, 1), :] = logp

        ndx = jnp.sum(onehot * rawx_ref[...], axis=0, keepdims=True)
        ndy = jnp.sum(onehot * rawy_ref[...], axis=0, keepdims=True)
        return (ndx, ndy)

    jax.lax.fori_loop(0, T, step, (fi_ref[0:1, :], fi_ref[1:2, :]))


def kernel(raw_features, first_input, W_s, b_s, W_d, b_d, Wi, Wh, bi, bh, W_ref, W_q, v):
    B, N, F = raw_features.shape
    H = W_s.shape[1]
    T = N - 1

    rawxT = raw_features[:, :, 0].T
    rawyT = raw_features[:, :, 1].T
    rawxF = rawxT.reshape(1, N * B)
    rawyF = rawyT.reshape(1, N * B)
    fi = jnp.broadcast_to(first_input[0, 0, :][:, None], (F, B)).astype(_f32)

    out_idx, out_logp = pl.pallas_call(
        _decode_body,
        out_shape=[
            jax.ShapeDtypeStruct((T, B), _f32),
            jax.ShapeDtypeStruct((T, B), _f32),
        ],
        scratch_shapes=[
            pltpu.VMEM((H, N * B), _f32),
            pltpu.VMEM((H, B), _f32),
            pltpu.VMEM((N, B), _f32),
        ],
    )(rawxT, rawyT, rawxF, rawyF, fi, W_s[0][:, None], W_s[1][:, None], b_s[:, None],
      W_d[0][:, None], W_d[1][:, None], b_d[:, None], Wi.T, Wh.T,
      bi[:, None], bh[:, None], W_ref.T, W_q.T, v[None, :])

    return (out_idx.T, out_logp.T)


# speedup vs baseline: 1.0419x; 1.0356x over previous
import jax
import jax.numpy as jnp
from jax.experimental import pallas as pl
from jax.experimental.pallas import tpu as pltpu

_bf16 = jnp.bfloat16
_f32 = jnp.float32


def _bfp(a, b):
    return a.astype(_bf16).astype(_f32) * b.astype(_bf16).astype(_f32)


def _decode_body(rawx_ref, rawy_ref, rawxf_ref, rawyf_ref, fi_ref,
                 ws0_ref, ws1_ref, bs_ref,
                 wd0_ref, wd1_ref, bd_ref, wiT_ref, whT_ref, bi_ref, bh_ref,
                 wrefT_ref, wqT_ref, v_ref,
                 idx_ref, logp_ref,
                 rp_scr, h_scr, mask_scr):
    N, B = rawx_ref.shape
    H = h_scr.shape[0]
    T = N - 1

    wrefT_bf = wrefT_ref[...]
    ws0 = ws0_ref[...]
    ws1 = ws1_ref[...]
    bs = bs_ref[...]

    rxf = rawxf_ref[...]
    ryf = rawyf_ref[...]
    feat = _bfp(ws0, rxf) + _bfp(ws1, ryf) + bs
    rp_scr[...] = jnp.dot(wrefT_bf, feat, preferred_element_type=_f32)

    h_scr[...] = jnp.zeros((H, B), dtype=_f32)
    iota_i = jax.lax.broadcasted_iota(jnp.int32, (N, B), 0)
    iota_n = iota_i.astype(_f32)
    mask_scr[...] = jnp.where(iota_i == 0, 0.0, 1.0)

    wd0 = wd0_ref[...]
    wd1 = wd1_ref[...]
    bd = bd_ref[...]
    wiT_bf = wiT_ref[...]
    whT_bf = whT_ref[...]
    bi2 = bi_ref[...]
    bh2 = bh_ref[...]
    wqT_bf = wqT_ref[...]
    v_bf = v_ref[...]

    def step(t, dec):
        dx, dy = dec
        emb = _bfp(wd0, dx) + _bfp(wd1, dy) + bd
        hT = h_scr[...]
        gi = jnp.dot(wiT_bf, emb, preferred_element_type=_f32) + bi2
        gh = jnp.dot(whT_bf, hT, preferred_element_type=_f32) + bh2
        r = jax.nn.sigmoid(gi[0:H] + gh[0:H])
        z = jax.nn.sigmoid(gi[H:2 * H] + gh[H:2 * H])
        n_ = jnp.tanh(gi[2 * H:3 * H] + r * gh[2 * H:3 * H])
        hT = (1.0 - z) * n_ + z * hT
        h_scr[...] = hT

        qT = jnp.dot(wqT_bf, hT, preferred_element_type=_f32)

        q_rep = jnp.concatenate([qT] * 8, axis=1)
        rows = []
        for g in range(0, N, 8):
            w = min(8, N - g)
            tt = jnp.tanh(rp_scr[:, g * B:(g + w) * B] + q_rep[:, :w * B])
            lg = jnp.dot(v_bf, tt, preferred_element_type=_f32)
            rows.extend(lg[:, j * B:(j + 1) * B] for j in range(w))
        logits = jnp.concatenate(rows, axis=0)

        maskv = mask_scr[...]
        masked = jnp.where(maskv > 0.0, logits, -jnp.inf)
        m = jnp.max(masked, axis=0, keepdims=True)
        unnorm = jnp.exp(masked - m)
        s = jnp.sum(unnorm, axis=0, keepdims=True)
        probs = unnorm / s
        prob = jnp.max(probs, axis=0, keepdims=True)
        logp = jnp.log(prob)

        eq = probs == prob
        chosen = jnp.min(jnp.where(eq, iota_n, 1e9), axis=0, keepdims=True)
        onehot = (iota_n == chosen).astype(_f32)
        mask_scr[...] = maskv * (1.0 - onehot)

        idx_ref[pl.ds(t, 1), :] = chosen
        logp_ref[pl.ds(t, 1), :] = logp

        ndx = jnp.sum(onehot * rawx_ref[...], axis=0, keepdims=True)
        ndy = jnp.sum(onehot * rawy_ref[...], axis=0, keepdims=True)
        return (ndx, ndy)

    jax.lax.fori_loop(0, T, step, (fi_ref[0:1, :], fi_ref[1:2, :]), unroll=7)


def kernel(raw_features, first_input, W_s, b_s, W_d, b_d, Wi, Wh, bi, bh, W_ref, W_q, v):
    B, N, F = raw_features.shape
    H = W_s.shape[1]
    T = N - 1

    rawxT = raw_features[:, :, 0].T
    rawyT = raw_features[:, :, 1].T
    rawxF = rawxT.reshape(1, N * B)
    rawyF = rawyT.reshape(1, N * B)
    fi = jnp.broadcast_to(first_input[0, 0, :][:, None], (F, B)).astype(_f32)

    out_idx, out_logp = pl.pallas_call(
        _decode_body,
        out_shape=[
            jax.ShapeDtypeStruct((T, B), _f32),
            jax.ShapeDtypeStruct((T, B), _f32),
        ],
        scratch_shapes=[
            pltpu.VMEM((H, N * B), _f32),
            pltpu.VMEM((H, B), _f32),
            pltpu.VMEM((N, B), _f32),
        ],
    )(rawxT, rawyT, rawxF, rawyF, fi, W_s[0][:, None], W_s[1][:, None], b_s[:, None],
      W_d[0][:, None], W_d[1][:, None], b_d[:, None], Wi.T, Wh.T,
      bi[:, None], bh[:, None], W_ref.T, W_q.T, v[None, :])

    return (out_idx.T, out_logp.T)
